# baseline (speedup 1.0000x reference)
import functools

import jax
import jax.numpy as jnp
from jax import lax
from jax.experimental import pallas as pl
from jax.experimental.pallas import tpu as pltpu
from jax.experimental.pallas import tpu_sc as plsc

N = 10000
E = 320000
D = 128

NC = 2
NS = 16
LANES = 16
NW = NC * NS

N_PAD = 10240
ROWS_S = N_PAD // NS
BROWS = N_PAD // NW
E_PAD = 327680
EW = E_PAD // NW
CHUNK = 128
GPC = CHUNK // LANES
SAFE_SRC = N
SAFE_REC = SAFE_SRC << 14
TRASH = 333824
SORT_LEN = TRASH + NW * CHUNK

_mesh = plsc.VectorSubcoreMesh(core_axis_name="c", subcore_axis_name="s",
                               num_cores=NC, num_subcores=NS)
_sc_params = pltpu.CompilerParams(needs_layout_passes=False)


@functools.partial(
    pl.kernel,
    out_type=(
        jax.ShapeDtypeStruct((NC, N_PAD), jnp.int32),
        jax.ShapeDtypeStruct((NW, NW), jnp.int32),
    ),
    mesh=_mesh,
    scratch_types=[
        pltpu.VMEM((N_PAD,), jnp.int32),
        pltpu.VMEM((EW,), jnp.int32),
        pltpu.VMEM((NS, ROWS_S), jnp.int32),
        pltpu.VMEM((ROWS_S,), jnp.int32),
        pltpu.VMEM((NW,), jnp.int32),
        pltpu.VMEM_SHARED((NS, N_PAD), jnp.int32),
    ],
    compiler_params=_sc_params,
)
def _deg_kernel(dst_hbm, out_hbm, c_hbm, hist, dstbuf, redbuf, resbuf, cbuf,
                stage):
    c = lax.axis_index("c")
    s = lax.axis_index("s")
    wid = s * NC + c

    zeros16 = jnp.zeros((LANES,), jnp.int32)
    ones16 = jnp.ones((LANES,), jnp.int32)
    iota16 = lax.iota(jnp.int32, LANES)

    def zero_body(i, carry):
        hist[pl.ds(i * LANES, LANES)] = zeros16
        return carry

    lax.fori_loop(0, N_PAD // LANES, zero_body, 0)

    pltpu.sync_copy(dst_hbm.at[pl.ds(wid * EW, EW)], dstbuf)

    def hist_body(i, carry):
        idx = dstbuf[pl.ds(i * LANES, LANES)]
        plsc.addupdate_scatter(hist, [idx], ones16)
        return carry

    lax.fori_loop(0, EW // LANES, hist_body, 0)

    def csum_body(j, carry):
        v0, v1 = carry
        v0 = v0 + plsc.load_gather(hist, [iota16 * BROWS + j])
        v1 = v1 + plsc.load_gather(hist, [LANES * BROWS + iota16 * BROWS + j])
        return v0, v1

    v0, v1 = lax.fori_loop(0, BROWS, csum_body, (zeros16, zeros16))
    cbuf[pl.ds(0, LANES)] = v0
    cbuf[pl.ds(LANES, LANES)] = v1
    pltpu.sync_copy(cbuf, c_hbm.at[wid])

    pltpu.sync_copy(hist, stage.at[s])
    plsc.subcore_barrier()

    def fetch_body(i, carry):
        pltpu.sync_copy(stage.at[i, pl.ds(s * ROWS_S, ROWS_S)], redbuf.at[i])
        return carry

    lax.fori_loop(0, NS, fetch_body, 0)

    def reduce_body(i, carry):
        v = redbuf[0, pl.ds(i * LANES, LANES)]
        for j in range(1, NS):
            v = v + redbuf[j, pl.ds(i * LANES, LANES)]
        resbuf[pl.ds(i * LANES, LANES)] = v
        return carry

    lax.fori_loop(0, ROWS_S // LANES, reduce_body, 0)

    pltpu.sync_copy(resbuf, out_hbm.at[c, pl.ds(s * ROWS_S, ROWS_S)])


def _prescale_body(hist_ref, x_ref, o_ref):
    deg = (hist_ref[..., 0] + hist_ref[..., 1] + 1).astype(jnp.float32)
    dinv = lax.rsqrt(deg)
    o_ref[...] = x_ref[...] * dinv[:, None]


_B2 = 1024


def _prescale(hist_t, x_pad):
    return pl.pallas_call(
        _prescale_body,
        grid=(N_PAD // _B2,),
        in_specs=[
            pl.BlockSpec((_B2, 2), lambda i: (i, 0)),
            pl.BlockSpec((_B2, D), lambda i: (i, 0)),
        ],
        out_specs=pl.BlockSpec((_B2, D), lambda i: (i, 0)),
        out_shape=jax.ShapeDtypeStruct((N_PAD, D), jnp.float32),
    )(hist_t, x_pad)


@functools.partial(
    pl.kernel,
    out_type=jax.ShapeDtypeStruct((SORT_LEN,), jnp.int32),
    mesh=_mesh,
    scratch_types=[
        pltpu.VMEM((EW,), jnp.int32),
        pltpu.VMEM((EW,), jnp.int32),
        pltpu.VMEM((NW,), jnp.int32),
        pltpu.VMEM((LANES,), jnp.int32),
        pltpu.VMEM((CHUNK,), jnp.int32),
        pltpu.VMEM((CHUNK,), jnp.int32),
        pltpu.VMEM((NW + NW + 1,), jnp.int32),
    ],
    compiler_params=_sc_params,
)
def _sort_kernel(src_hbm, dst_hbm, p_hbm, meta_hbm, out_hbm,
                 srcb, dstb, nextpos, ksbuf, posb, recb, metab):
    c = lax.axis_index("c")
    s = lax.axis_index("s")
    wid = s * NC + c

    iota16 = lax.iota(jnp.int32, LANES)
    safe16 = jnp.full((LANES,), SAFE_REC, jnp.int32)

    pltpu.sync_copy(meta_hbm, metab)
    pltpu.sync_copy(src_hbm.at[pl.ds(wid * EW, EW)], srcb)
    pltpu.sync_copy(dst_hbm.at[pl.ds(wid * EW, EW)], dstb)
    pltpu.sync_copy(p_hbm.at[wid], nextpos)

    wid16 = jnp.full((LANES,), wid, jnp.int32)
    gap_lo = (plsc.load_gather(metab, [wid16])
              + plsc.load_gather(metab, [wid16 + (NW + 1)]))
    gap_hi = plsc.load_gather(metab, [wid16 + 1])
    for g in range(GPC):
        lane = iota16 + g * LANES
        pos = gap_lo + lane
        pos = jnp.where(pos < gap_hi, pos, TRASH + wid * CHUNK + lane)
        posb[pl.ds(g * LANES, LANES)] = pos
        recb[pl.ds(g * LANES, LANES)] = safe16
    pltpu.sync_copy(recb, out_hbm.at[posb])

    def chunk_body(o, carry):
        for g in range(GPC):
            off = o * CHUNK + g * LANES
            sv = srcb[pl.ds(off, LANES)]
            dv = dstb[pl.ds(off, LANES)]
            kv = lax.div(dv, BROWS)
            rec = (sv << 14) | dv
            ks, vs = plsc.sort_key_val(kv, rec)
            ksbuf[...] = ks
            prev = plsc.load_gather(ksbuf, [jnp.maximum(iota16 - 1, 0)])
            nxt = plsc.load_gather(ksbuf, [jnp.minimum(iota16 + 1, LANES - 1)])
            is_start = (iota16 == 0) | (ks != prev)
            is_end = (iota16 == LANES - 1) | (ks != nxt)
            start_idx = plsc.cummax(jnp.where(is_start, iota16, 0))
            rank = iota16 - start_idx
            base = plsc.load_gather(nextpos, [ks])
            posb[pl.ds(g * LANES, LANES)] = base + rank
            recb[pl.ds(g * LANES, LANES)] = vs
            plsc.addupdate_scatter(nextpos, [ks], rank + 1, mask=is_end)
        pltpu.sync_copy(recb, out_hbm.at[posb])
        return carry

    lax.fori_loop(0, EW // CHUNK, chunk_body, 0)


@functools.partial(
    pl.kernel,
    out_type=jax.ShapeDtypeStruct((N_PAD, D), jnp.float32),
    mesh=_mesh,
    scratch_types=[
        pltpu.VMEM((CHUNK,), jnp.int32),
        pltpu.VMEM((CHUNK,), jnp.int32),
        pltpu.VMEM((CHUNK,), jnp.int32),
        pltpu.VMEM((CHUNK,), jnp.int32),
        pltpu.VMEM((CHUNK,), jnp.int32),
        pltpu.VMEM((CHUNK, D), jnp.float32),
        pltpu.VMEM((CHUNK, D), jnp.float32),
        pltpu.VMEM((BROWS, D), jnp.float32),
        pltpu.VMEM((NW + NW + 1,), jnp.int32),
        pltpu.SemaphoreType.DMA,
        pltpu.SemaphoreType.DMA,
    ],
    compiler_params=_sc_params,
)
def _agg_kernel(x2_hbm, recs_hbm, meta_hbm, out_hbm,
                recb, sidx0, sidx1, dstl0, dstl1, rows0, rows1, accl, metab,
                sem0, sem1):
    c = lax.axis_index("c")
    s = lax.axis_index("s")
    wid = s * NC + c

    iota16 = lax.iota(jnp.int32, LANES)
    zeros16 = jnp.zeros((LANES,), jnp.float32)

    pltpu.sync_copy(meta_hbm, metab)
    wid16 = jnp.full((LANES,), wid, jnp.int32)
    s0 = lax.reduce_max(plsc.load_gather(metab, [wid16]), axes=(0,))
    s1 = lax.reduce_max(plsc.load_gather(metab, [wid16 + 1]), axes=(0,))
    s0 = pl.multiple_of(s0, CHUNK)
    nch = (s1 - s0) >> 7

    def zero_body(i, carry):
        accl[i // (D // LANES), pl.ds((i % (D // LANES)) * LANES, LANES)] = (
            zeros16)
        return carry

    lax.fori_loop(0, BROWS * (D // LANES), zero_body, 0)

    dbase = wid * BROWS

    def load_idx(ch, sidx, dstl):
        pltpu.sync_copy(recs_hbm.at[pl.ds(s0 + ch * CHUNK, CHUNK)], recb)
        for g in range(GPC):
            rv = recb[pl.ds(g * LANES, LANES)]
            sidx[pl.ds(g * LANES, LANES)] = rv >> 14
            dl = (rv & (16384 - 1)) - dbase
            dstl[pl.ds(g * LANES, LANES)] = jnp.clip(dl, 0, BROWS - 1)

    def accumulate(rows, dstl):
        for g in range(GPC):
            dl = dstl[pl.ds(g * LANES, LANES)]
            rbase = iota16 + g * LANES

            def col_body(cb, carry):
                for u in range(8):
                    col = cb * 8 + u
                    colv = jnp.full((LANES,), col, jnp.int32)
                    vals = plsc.load_gather(rows, [rbase, colv])
                    plsc.addupdate_scatter(accl, [dl, colv], vals)
                return carry

            lax.fori_loop(0, D // 8, col_body, 0)

    @pl.when(nch > 0)
    def _():
        load_idx(0, sidx0, dstl0)
        pltpu.async_copy(x2_hbm.at[sidx0], rows0, sem0)

        def chunk_body(j, carry):
            even = lax.rem(j, 2) == 0

            @pl.when(even)
            def _():
                @pl.when(j + 1 < nch)
                def _():
                    load_idx(j + 1, sidx1, dstl1)
                    pltpu.async_copy(x2_hbm.at[sidx1], rows1, sem1)

                pltpu.make_async_copy(x2_hbm.at[sidx0], rows0, sem0).wait()
                accumulate(rows0, dstl0)

            @pl.when(jnp.logical_not(even))
            def _():
                @pl.when(j + 1 < nch)
                def _():
                    load_idx(j + 1, sidx0, dstl0)
                    pltpu.async_copy(x2_hbm.at[sidx0], rows0, sem0)

                pltpu.make_async_copy(x2_hbm.at[sidx1], rows1, sem1).wait()
                accumulate(rows1, dstl1)

            return carry

        lax.fori_loop(0, nch, chunk_body, 0)

    pltpu.sync_copy(accl, out_hbm.at[pl.ds(dbase, BROWS), :])


def _out_body(hist_ref, acc_ref, x2_ref, w_ref, b_ref, a_ref, o_ref):
    deg = (hist_ref[..., 0] + hist_ref[..., 1] + 1).astype(jnp.float32)
    dinv = lax.rsqrt(deg)
    agg = (acc_ref[...] + x2_ref[...]) * dinv[:, None]
    h = jnp.dot(agg, w_ref[...], preferred_element_type=jnp.float32)
    h = h + b_ref[...]
    o_ref[...] = jnp.where(h > 0, h, a_ref[...] * h)


_B4 = 512


def _finalize(hist_t, acc, x2, W, b2, a2):
    return pl.pallas_call(
        _out_body,
        grid=(N_PAD // _B4,),
        in_specs=[
            pl.BlockSpec((_B4, 2), lambda i: (i, 0)),
            pl.BlockSpec((_B4, D), lambda i: (i, 0)),
            pl.BlockSpec((_B4, D), lambda i: (i, 0)),
            pl.BlockSpec((D, D), lambda i: (0, 0)),
            pl.BlockSpec((1, D), lambda i: (0, 0)),
            pl.BlockSpec((1, D), lambda i: (0, 0)),
        ],
        out_specs=pl.BlockSpec((_B4, D), lambda i: (i, 0)),
        out_shape=jax.ShapeDtypeStruct((N_PAD, D), jnp.float32),
    )(hist_t, acc, x2, W, b2, a2)


def kernel(x, edge_index, edge_type, W, b, a):
    del edge_type
    src = edge_index[0].astype(jnp.int32)
    dst = edge_index[1].astype(jnp.int32)
    src_p = jnp.concatenate([src, jnp.full((E_PAD - E,), SAFE_SRC, jnp.int32)])
    dst_p = jnp.concatenate([dst, jnp.full((E_PAD - E,), N_PAD - 1, jnp.int32)])
    x_pad = jnp.zeros((N_PAD, D), jnp.float32).at[:N].set(x)

    hist, cnt = _deg_kernel(dst_p)
    hist_t = hist.T

    tot = cnt.sum(axis=0)
    sizes = ((tot + CHUNK - 1) // CHUNK) * CHUNK
    start = jnp.concatenate(
        [jnp.zeros((1,), jnp.int32), jnp.cumsum(sizes).astype(jnp.int32)])
    p_mat = start[None, :NW] + (jnp.cumsum(cnt, axis=0) - cnt)
    meta = jnp.concatenate([start, tot]).astype(jnp.int32)

    x2 = _prescale(hist_t, x_pad)
    recs = _sort_kernel(src_p, dst_p, p_mat.astype(jnp.int32), meta)
    acc = _agg_kernel(x2, recs, meta)
    out = _finalize(hist_t, acc, x2, W,
                    b.reshape(1, D), a.reshape(1, D))
    return out[:N]


# speedup vs baseline: 1.1880x; 1.1880x over previous
import functools

import jax
import jax.numpy as jnp
from jax import lax
from jax.experimental import pallas as pl
from jax.experimental.pallas import tpu as pltpu
from jax.experimental.pallas import tpu_sc as plsc

N = 10000
E = 320000
D = 128

NC = 2
NS = 16
LANES = 16
NW = NC * NS

N_PAD = 10240
ROWS_S = N_PAD // NS
BROWS = N_PAD // NW
E_PAD = 327680
EW = E_PAD // NW
CHUNK = 128
GPC = CHUNK // LANES
SAFE_SRC = N
SAFE_REC = SAFE_SRC << 14
TRASH = 333824
SORT_LEN = TRASH + NW * CHUNK

_mesh = plsc.VectorSubcoreMesh(core_axis_name="c", subcore_axis_name="s",
                               num_cores=NC, num_subcores=NS)
_sc_params = pltpu.CompilerParams(needs_layout_passes=False)


@functools.partial(
    pl.kernel,
    out_type=(
        jax.ShapeDtypeStruct((NC, N_PAD), jnp.int32),
        jax.ShapeDtypeStruct((NW, NW), jnp.int32),
    ),
    mesh=_mesh,
    scratch_types=[
        pltpu.VMEM((N_PAD,), jnp.int32),
        pltpu.VMEM((EW,), jnp.int32),
        pltpu.VMEM((NS, ROWS_S), jnp.int32),
        pltpu.VMEM((ROWS_S,), jnp.int32),
        pltpu.VMEM((NW,), jnp.int32),
        pltpu.VMEM_SHARED((NS, N_PAD), jnp.int32),
    ],
    compiler_params=_sc_params,
)
def _deg_kernel(dst_hbm, out_hbm, c_hbm, hist, dstbuf, redbuf, resbuf, cbuf,
                stage):
    c = lax.axis_index("c")
    s = lax.axis_index("s")
    wid = s * NC + c

    zeros16 = jnp.zeros((LANES,), jnp.int32)
    ones16 = jnp.ones((LANES,), jnp.int32)
    iota16 = lax.iota(jnp.int32, LANES)

    def zero_body(i, carry):
        hist[pl.ds(i * LANES, LANES)] = zeros16
        return carry

    lax.fori_loop(0, N_PAD // LANES, zero_body, 0)

    pltpu.sync_copy(dst_hbm.at[pl.ds(wid * EW, EW)], dstbuf)

    def hist_body(i, carry):
        idx = dstbuf[pl.ds(i * LANES, LANES)]
        plsc.addupdate_scatter(hist, [idx], ones16)
        return carry

    lax.fori_loop(0, EW // LANES, hist_body, 0)

    def csum_body(j, carry):
        v0, v1 = carry
        v0 = v0 + plsc.load_gather(hist, [iota16 * BROWS + j])
        v1 = v1 + plsc.load_gather(hist, [LANES * BROWS + iota16 * BROWS + j])
        return v0, v1

    v0, v1 = lax.fori_loop(0, BROWS, csum_body, (zeros16, zeros16))
    cbuf[pl.ds(0, LANES)] = v0
    cbuf[pl.ds(LANES, LANES)] = v1
    pltpu.sync_copy(cbuf, c_hbm.at[wid])

    pltpu.sync_copy(hist, stage.at[s])
    plsc.subcore_barrier()

    def fetch_body(i, carry):
        pltpu.sync_copy(stage.at[i, pl.ds(s * ROWS_S, ROWS_S)], redbuf.at[i])
        return carry

    lax.fori_loop(0, NS, fetch_body, 0)

    def reduce_body(i, carry):
        v = redbuf[0, pl.ds(i * LANES, LANES)]
        for j in range(1, NS):
            v = v + redbuf[j, pl.ds(i * LANES, LANES)]
        resbuf[pl.ds(i * LANES, LANES)] = v
        return carry

    lax.fori_loop(0, ROWS_S // LANES, reduce_body, 0)

    pltpu.sync_copy(resbuf, out_hbm.at[c, pl.ds(s * ROWS_S, ROWS_S)])


def _prescale_body(hist_ref, x_ref, o_ref):
    deg = (hist_ref[..., 0] + hist_ref[..., 1] + 1).astype(jnp.float32)
    dinv = lax.rsqrt(deg)
    o_ref[...] = x_ref[...] * dinv[:, None]


_B2 = 1024


def _prescale(hist_t, x_pad):
    return pl.pallas_call(
        _prescale_body,
        grid=(N_PAD // _B2,),
        in_specs=[
            pl.BlockSpec((_B2, 2), lambda i: (i, 0)),
            pl.BlockSpec((_B2, D), lambda i: (i, 0)),
        ],
        out_specs=pl.BlockSpec((_B2, D), lambda i: (i, 0)),
        out_shape=jax.ShapeDtypeStruct((N_PAD, D), jnp.float32),
    )(hist_t, x_pad)


@functools.partial(
    pl.kernel,
    out_type=jax.ShapeDtypeStruct((SORT_LEN,), jnp.int32),
    mesh=_mesh,
    scratch_types=[
        pltpu.VMEM((EW,), jnp.int32),
        pltpu.VMEM((EW,), jnp.int32),
        pltpu.VMEM((NW,), jnp.int32),
        pltpu.VMEM((LANES,), jnp.int32),
        pltpu.VMEM((CHUNK,), jnp.int32),
        pltpu.VMEM((CHUNK,), jnp.int32),
        pltpu.VMEM((CHUNK,), jnp.int32),
        pltpu.VMEM((CHUNK,), jnp.int32),
        pltpu.VMEM((NW + NW + 1,), jnp.int32),
        pltpu.SemaphoreType.DMA,
        pltpu.SemaphoreType.DMA,
    ],
    compiler_params=_sc_params,
)
def _sort_kernel(src_hbm, dst_hbm, p_hbm, meta_hbm, out_hbm,
                 srcb, dstb, nextpos, ksbuf, posb0, recb0, posb1, recb1,
                 metab, sem_a, sem_b):
    c = lax.axis_index("c")
    s = lax.axis_index("s")
    wid = s * NC + c

    iota16 = lax.iota(jnp.int32, LANES)
    safe16 = jnp.full((LANES,), SAFE_REC, jnp.int32)

    pltpu.sync_copy(meta_hbm, metab)
    pltpu.sync_copy(src_hbm.at[pl.ds(wid * EW, EW)], srcb)
    pltpu.sync_copy(dst_hbm.at[pl.ds(wid * EW, EW)], dstb)
    pltpu.sync_copy(p_hbm.at[wid], nextpos)

    wid16 = jnp.full((LANES,), wid, jnp.int32)
    gap_lo = (plsc.load_gather(metab, [wid16])
              + plsc.load_gather(metab, [wid16 + (NW + 1)]))
    gap_hi = plsc.load_gather(metab, [wid16 + 1])
    for g in range(GPC):
        lane = iota16 + g * LANES
        pos = gap_lo + lane
        pos = jnp.where(pos < gap_hi, pos, TRASH + wid * CHUNK + lane)
        posb1[pl.ds(g * LANES, LANES)] = pos
        recb1[pl.ds(g * LANES, LANES)] = safe16
    pltpu.async_copy(recb1, out_hbm.at[posb1], sem_b)

    def fill(o, posb, recb):
        for g in range(GPC):
            off = o * CHUNK + g * LANES
            sv = srcb[pl.ds(off, LANES)]
            dv = dstb[pl.ds(off, LANES)]
            kv = lax.div(dv, BROWS)
            rec = (sv << 14) | dv
            ks, vs = plsc.sort_key_val(kv, rec)
            ksbuf[...] = ks
            prev = plsc.load_gather(ksbuf, [jnp.maximum(iota16 - 1, 0)])
            nxt = plsc.load_gather(ksbuf, [jnp.minimum(iota16 + 1, LANES - 1)])
            is_start = (iota16 == 0) | (ks != prev)
            is_end = (iota16 == LANES - 1) | (ks != nxt)
            start_idx = plsc.cummax(jnp.where(is_start, iota16, 0))
            rank = iota16 - start_idx
            base = plsc.load_gather(nextpos, [ks])
            posb[pl.ds(g * LANES, LANES)] = base + rank
            recb[pl.ds(g * LANES, LANES)] = vs
            plsc.addupdate_scatter(nextpos, [ks], rank + 1, mask=is_end)

    def chunk_body(o, carry):
        even = lax.rem(o, 2) == 0

        @pl.when(even)
        def _():
            @pl.when(o >= 2)
            def _():
                pltpu.make_async_copy(recb0, out_hbm.at[posb0], sem_a).wait()

            fill(o, posb0, recb0)
            pltpu.async_copy(recb0, out_hbm.at[posb0], sem_a)

        @pl.when(jnp.logical_not(even))
        def _():
            pltpu.make_async_copy(recb1, out_hbm.at[posb1], sem_b).wait()
            fill(o, posb1, recb1)
            pltpu.async_copy(recb1, out_hbm.at[posb1], sem_b)

        return carry

    lax.fori_loop(0, EW // CHUNK, chunk_body, 0)
    pltpu.make_async_copy(recb0, out_hbm.at[posb0], sem_a).wait()
    pltpu.make_async_copy(recb1, out_hbm.at[posb1], sem_b).wait()


@functools.partial(
    pl.kernel,
    out_type=jax.ShapeDtypeStruct((N_PAD, D), jnp.float32),
    mesh=_mesh,
    scratch_types=[
        pltpu.VMEM((CHUNK,), jnp.int32),
        pltpu.VMEM((CHUNK,), jnp.int32),
        pltpu.VMEM((CHUNK,), jnp.int32),
        pltpu.VMEM((CHUNK,), jnp.int32),
        pltpu.VMEM((CHUNK,), jnp.int32),
        pltpu.VMEM((CHUNK, D), jnp.float32),
        pltpu.VMEM((CHUNK, D), jnp.float32),
        pltpu.VMEM((BROWS, D), jnp.float32),
        pltpu.VMEM((NW + NW + 1,), jnp.int32),
        pltpu.SemaphoreType.DMA,
        pltpu.SemaphoreType.DMA,
    ],
    compiler_params=_sc_params,
)
def _agg_kernel(x2_hbm, recs_hbm, meta_hbm, out_hbm,
                recb, sidx0, sidx1, dstl0, dstl1, rows0, rows1, accl, metab,
                sem0, sem1):
    c = lax.axis_index("c")
    s = lax.axis_index("s")
    wid = s * NC + c

    iota16 = lax.iota(jnp.int32, LANES)
    zeros16 = jnp.zeros((LANES,), jnp.float32)

    pltpu.sync_copy(meta_hbm, metab)
    wid16 = jnp.full((LANES,), wid, jnp.int32)
    s0 = lax.reduce_max(plsc.load_gather(metab, [wid16]), axes=(0,))
    s1 = lax.reduce_max(plsc.load_gather(metab, [wid16 + 1]), axes=(0,))
    s0 = pl.multiple_of(s0, CHUNK)
    nch = (s1 - s0) >> 7

    def zero_body(i, carry):
        accl[i // (D // LANES), pl.ds((i % (D // LANES)) * LANES, LANES)] = (
            zeros16)
        return carry

    lax.fori_loop(0, BROWS * (D // LANES), zero_body, 0)

    dbase = wid * BROWS

    def load_idx(ch, sidx, dstl):
        pltpu.sync_copy(recs_hbm.at[pl.ds(s0 + ch * CHUNK, CHUNK)], recb)
        for g in range(GPC):
            rv = recb[pl.ds(g * LANES, LANES)]
            sidx[pl.ds(g * LANES, LANES)] = rv >> 14
            dl = (rv & (16384 - 1)) - dbase
            dstl[pl.ds(g * LANES, LANES)] = jnp.clip(dl, 0, BROWS - 1)

    def accumulate(rows, dstl):
        for g in range(GPC):
            dl = dstl[pl.ds(g * LANES, LANES)]
            rbase = iota16 + g * LANES

            def col_body(cb, carry):
                cols = [cb * LANES + u for u in range(LANES)]
                colvs = [jnp.full((LANES,), col, jnp.int32) for col in cols]
                vals = [plsc.load_gather(rows, [rbase, cv]) for cv in colvs]
                for cv, v in zip(colvs, vals):
                    plsc.addupdate_scatter(accl, [dl, cv], v)
                return carry

            lax.fori_loop(0, D // LANES, col_body, 0)

    @pl.when(nch > 0)
    def _():
        load_idx(0, sidx0, dstl0)
        pltpu.async_copy(x2_hbm.at[sidx0], rows0, sem0)

        def chunk_body(j, carry):
            even = lax.rem(j, 2) == 0

            @pl.when(even)
            def _():
                @pl.when(j + 1 < nch)
                def _():
                    load_idx(j + 1, sidx1, dstl1)
                    pltpu.async_copy(x2_hbm.at[sidx1], rows1, sem1)

                pltpu.make_async_copy(x2_hbm.at[sidx0], rows0, sem0).wait()
                accumulate(rows0, dstl0)

            @pl.when(jnp.logical_not(even))
            def _():
                @pl.when(j + 1 < nch)
                def _():
                    load_idx(j + 1, sidx0, dstl0)
                    pltpu.async_copy(x2_hbm.at[sidx0], rows0, sem0)

                pltpu.make_async_copy(x2_hbm.at[sidx1], rows1, sem1).wait()
                accumulate(rows1, dstl1)

            return carry

        lax.fori_loop(0, nch, chunk_body, 0)

    pltpu.sync_copy(accl, out_hbm.at[pl.ds(dbase, BROWS), :])


def _out_body(hist_ref, acc_ref, x2_ref, w_ref, b_ref, a_ref, o_ref):
    deg = (hist_ref[..., 0] + hist_ref[..., 1] + 1).astype(jnp.float32)
    dinv = lax.rsqrt(deg)
    agg = (acc_ref[...] + x2_ref[...]) * dinv[:, None]
    h = jnp.dot(agg, w_ref[...], preferred_element_type=jnp.float32)
    h = h + b_ref[...]
    o_ref[...] = jnp.where(h > 0, h, a_ref[...] * h)


_B4 = 512


def _finalize(hist_t, acc, x2, W, b2, a2):
    return pl.pallas_call(
        _out_body,
        grid=(N_PAD // _B4,),
        in_specs=[
            pl.BlockSpec((_B4, 2), lambda i: (i, 0)),
            pl.BlockSpec((_B4, D), lambda i: (i, 0)),
            pl.BlockSpec((_B4, D), lambda i: (i, 0)),
            pl.BlockSpec((D, D), lambda i: (0, 0)),
            pl.BlockSpec((1, D), lambda i: (0, 0)),
            pl.BlockSpec((1, D), lambda i: (0, 0)),
        ],
        out_specs=pl.BlockSpec((_B4, D), lambda i: (i, 0)),
        out_shape=jax.ShapeDtypeStruct((N_PAD, D), jnp.float32),
    )(hist_t, acc, x2, W, b2, a2)


def kernel(x, edge_index, edge_type, W, b, a):
    del edge_type
    src = edge_index[0].astype(jnp.int32)
    dst = edge_index[1].astype(jnp.int32)
    src_p = jnp.concatenate([src, jnp.full((E_PAD - E,), SAFE_SRC, jnp.int32)])
    dst_p = jnp.concatenate([dst, jnp.full((E_PAD - E,), N_PAD - 1, jnp.int32)])
    x_pad = jnp.zeros((N_PAD, D), jnp.float32).at[:N].set(x)

    hist, cnt = _deg_kernel(dst_p)
    hist_t = hist.T

    tot = cnt.sum(axis=0)
    sizes = ((tot + CHUNK - 1) // CHUNK) * CHUNK
    start = jnp.concatenate(
        [jnp.zeros((1,), jnp.int32), jnp.cumsum(sizes).astype(jnp.int32)])
    p_mat = start[None, :NW] + (jnp.cumsum(cnt, axis=0) - cnt)
    meta = jnp.concatenate([start, tot]).astype(jnp.int32)

    x2 = _prescale(hist_t, x_pad)
    recs = _sort_kernel(src_p, dst_p, p_mat.astype(jnp.int32), meta)
    acc = _agg_kernel(x2, recs, meta)
    out = _finalize(hist_t, acc, x2, W,
                    b.reshape(1, D), a.reshape(1, D))
    return out[:N]


# speedup vs baseline: 3.9438x; 3.3196x over previous
import functools

import jax
import jax.numpy as jnp
from jax import lax
from jax.experimental import pallas as pl
from jax.experimental.pallas import tpu as pltpu
from jax.experimental.pallas import tpu_sc as plsc

N = 10000
E = 320000
D = 128

NC = 2
NS = 16
LANES = 16
NW = NC * NS

N_PAD = 10240
ROWS_S = N_PAD // NS
E_PAD = 327680
EW = E_PAD // NW
CHUNK = 128
NCHUNK = EW // CHUNK
ZROWS = 32
NB = NCHUNK // 2

_mesh = plsc.VectorSubcoreMesh(core_axis_name="c", subcore_axis_name="s",
                               num_cores=NC, num_subcores=NS)


@functools.partial(
    pl.kernel,
    out_type=jax.ShapeDtypeStruct((NC, N_PAD), jnp.int32),
    mesh=_mesh,
    scratch_types=[
        pltpu.VMEM((N_PAD,), jnp.int32),
        pltpu.VMEM((EW,), jnp.int32),
        pltpu.VMEM((NS, ROWS_S), jnp.int32),
        pltpu.VMEM((ROWS_S,), jnp.int32),
        pltpu.VMEM_SHARED((NS, N_PAD), jnp.int32),
    ],
    compiler_params=pltpu.CompilerParams(needs_layout_passes=False),
)
def _deg_kernel(dst_hbm, out_hbm, hist, dstbuf, redbuf, resbuf, stage):
    c = lax.axis_index("c")
    s = lax.axis_index("s")
    wid = s * NC + c

    zeros16 = jnp.zeros((LANES,), jnp.int32)
    ones16 = jnp.ones((LANES,), jnp.int32)

    def zero_body(i, carry):
        hist[pl.ds(i * LANES, LANES)] = zeros16
        return carry

    lax.fori_loop(0, N_PAD // LANES, zero_body, 0)

    pltpu.sync_copy(dst_hbm.at[pl.ds(wid * EW, EW)], dstbuf)

    def hist_body(i, carry):
        idx = dstbuf[pl.ds(i * LANES, LANES)]
        plsc.addupdate_scatter(hist, [idx], ones16)
        return carry

    lax.fori_loop(0, EW // LANES, hist_body, 0)

    pltpu.sync_copy(hist, stage.at[s])
    plsc.subcore_barrier()

    def fetch_body(i, carry):
        pltpu.sync_copy(stage.at[i, pl.ds(s * ROWS_S, ROWS_S)], redbuf.at[i])
        return carry

    lax.fori_loop(0, NS, fetch_body, 0)

    def reduce_body(i, carry):
        v = redbuf[0, pl.ds(i * LANES, LANES)]
        for j in range(1, NS):
            v = v + redbuf[j, pl.ds(i * LANES, LANES)]
        resbuf[pl.ds(i * LANES, LANES)] = v
        return carry

    lax.fori_loop(0, ROWS_S // LANES, reduce_body, 0)

    pltpu.sync_copy(resbuf, out_hbm.at[c, pl.ds(s * ROWS_S, ROWS_S)])


def _prescale_body(hist_ref, x_ref, o_ref):
    deg = (hist_ref[..., 0] + hist_ref[..., 1] + 1).astype(jnp.float32)
    dinv = lax.rsqrt(deg)
    o_ref[...] = x_ref[...] * dinv[:, None]


_B2 = 1024


def _prescale(hist_t, x_pad):
    return pl.pallas_call(
        _prescale_body,
        grid=(N_PAD // _B2,),
        in_specs=[
            pl.BlockSpec((_B2, 2), lambda i: (i, 0)),
            pl.BlockSpec((_B2, D), lambda i: (i, 0)),
        ],
        out_specs=pl.BlockSpec((_B2, D), lambda i: (i, 0)),
        out_shape=jax.ShapeDtypeStruct((N_PAD, D), jnp.float32),
    )(hist_t, x_pad)


@functools.partial(
    pl.kernel,
    out_type=jax.ShapeDtypeStruct((NC, N_PAD, D), jnp.float32),
    mesh=_mesh,
    scratch_types=[
        pltpu.VMEM((NB, CHUNK), jnp.int32),
        pltpu.VMEM((NB, CHUNK), jnp.int32),
        pltpu.VMEM((CHUNK, D), jnp.float32),
        pltpu.VMEM((CHUNK, D), jnp.float32),
        pltpu.VMEM((ZROWS, D), jnp.float32),
        pltpu.VMEM_SHARED((N_PAD, D), jnp.float32),
        pltpu.SemaphoreType.DMA,
        pltpu.SemaphoreType.DMA,
    ],
)
def _agg_kernel(x2_hbm, src_hbm, dst_hbm, out_hbm,
                sidx, didx, rows0, rows1, zbuf, acc_s, sem0, sem1):
    c = lax.axis_index("c")
    s = lax.axis_index("s")
    wid = s * NC + c

    zeros16 = jnp.zeros((LANES,), jnp.float32)

    def zero_body(i, carry):
        zbuf[i // (D // LANES), pl.ds((i % (D // LANES)) * LANES, LANES)] = zeros16
        return carry

    lax.fori_loop(0, ZROWS * (D // LANES), zero_body, 0)

    base_row = s * ROWS_S
    for t in range(ROWS_S // ZROWS):
        pltpu.sync_copy(zbuf, acc_s.at[pl.ds(base_row + t * ZROWS, ZROWS), :])
    plsc.subcore_barrier()

    for half in range(2):
        cbase = wid * NCHUNK + half * NB
        pltpu.sync_copy(src_hbm.at[pl.ds(cbase, NB), :], sidx)
        pltpu.sync_copy(dst_hbm.at[pl.ds(cbase, NB), :], didx)
        pltpu.async_copy(x2_hbm.at[sidx.at[0]], rows0, sem0)

        def pair_body(jj, carry):
            j = jj * 2
            pltpu.async_copy(x2_hbm.at[sidx.at[j + 1]], rows1, sem1)
            pltpu.make_async_copy(x2_hbm.at[sidx.at[j]], rows0, sem0).wait()
            pltpu.sync_copy(rows0, acc_s.at[didx.at[j]], add=True)

            @pl.when(j + 2 < NB)
            def _():
                pltpu.async_copy(x2_hbm.at[sidx.at[j + 2]], rows0, sem0)

            pltpu.make_async_copy(x2_hbm.at[sidx.at[j + 1]], rows1, sem1).wait()
            pltpu.sync_copy(rows1, acc_s.at[didx.at[j + 1]], add=True)
            return carry

        lax.fori_loop(0, NB // 2, pair_body, 0)
    plsc.subcore_barrier()

    for t in range(ROWS_S // ZROWS):
        r0 = base_row + t * ZROWS
        pltpu.sync_copy(acc_s.at[pl.ds(r0, ZROWS), :], zbuf)
        pltpu.sync_copy(zbuf, out_hbm.at[c, pl.ds(r0, ZROWS), :])


def _out_body(hist_ref, a0_ref, a1_ref, x2_ref, w_ref, b_ref, a_ref, o_ref):
    deg = (hist_ref[..., 0] + hist_ref[..., 1] + 1).astype(jnp.float32)
    dinv = lax.rsqrt(deg)
    agg = (a0_ref[...] + a1_ref[...] + x2_ref[...]) * dinv[:, None]
    h = jnp.dot(agg, w_ref[...], preferred_element_type=jnp.float32)
    h = h + b_ref[...]
    o_ref[...] = jnp.where(h > 0, h, a_ref[...] * h)


_B4 = 512


def _finalize(hist_t, acc0, acc1, x2, W, b2, a2):
    return pl.pallas_call(
        _out_body,
        grid=(N_PAD // _B4,),
        in_specs=[
            pl.BlockSpec((_B4, 2), lambda i: (i, 0)),
            pl.BlockSpec((_B4, D), lambda i: (i, 0)),
            pl.BlockSpec((_B4, D), lambda i: (i, 0)),
            pl.BlockSpec((_B4, D), lambda i: (i, 0)),
            pl.BlockSpec((D, D), lambda i: (0, 0)),
            pl.BlockSpec((1, D), lambda i: (0, 0)),
            pl.BlockSpec((1, D), lambda i: (0, 0)),
        ],
        out_specs=pl.BlockSpec((_B4, D), lambda i: (i, 0)),
        out_shape=jax.ShapeDtypeStruct((N_PAD, D), jnp.float32),
    )(hist_t, acc0, acc1, x2, W, b2, a2)


def kernel(x, edge_index, edge_type, W, b, a):
    del edge_type
    src = edge_index[0].astype(jnp.int32)
    dst = edge_index[1].astype(jnp.int32)
    src_p = jnp.concatenate([src, jnp.full((E_PAD - E,), N, jnp.int32)])
    dst_p = jnp.concatenate([dst, jnp.full((E_PAD - E,), N_PAD - 1, jnp.int32)])
    src2d = src_p.reshape(NW * NCHUNK, CHUNK)
    dst2d = dst_p.reshape(NW * NCHUNK, CHUNK)
    x_pad = jnp.zeros((N_PAD, D), jnp.float32).at[:N].set(x)

    hist = _deg_kernel(dst_p)
    hist_t = hist.T
    x2 = _prescale(hist_t, x_pad)
    acc = _agg_kernel(x2, src2d, dst2d)
    out = _finalize(hist_t, acc[0], acc[1], x2, W,
                    b.reshape(1, D), a.reshape(1, D))
    return out[:N]
